# Initial kernel scaffold; baseline (speedup 1.0000x reference)
import jax
import jax.numpy as jnp
from jax import lax
from jax.experimental import pallas as pl
from jax.experimental.pallas import tpu as pltpu
from jax.experimental.pallas import tpu_sc as plsc

N = 10000
E = 320000
D = 128
DE = 16
H = 128
G = 64
DOUT = 1

NC = 2
NS = 16
NW = NC * NS
EPT = E // NW
CH = 80
NCHUNK = EPT // CH
RPT = N // NS

BN = 1000
NBLK = N // BN


def _sc_body(z_d, z_de, x_hbm, ei_hbm, ea_hbm, aggx_out, agge_out,
             src_v, dst_v, rows_v, ea_v, accx, acce, sem):
    c = lax.axis_index("c")
    s = lax.axis_index("s")
    wid = c * NS + s

    pltpu.sync_copy(z_d.at[pl.ds(s * RPT, RPT)], accx.at[pl.ds(s * RPT, RPT)])
    pltpu.sync_copy(z_de.at[pl.ds(s * RPT, RPT)], acce.at[pl.ds(s * RPT, RPT)])
    plsc.subcore_barrier()

    def body(j, carry):
        pltpu.sync_copy(ei_hbm.at[0, wid, j], src_v)
        pltpu.sync_copy(ei_hbm.at[1, wid, j], dst_v)
        pltpu.async_copy(x_hbm.at[src_v], rows_v, sem).wait()
        pltpu.sync_copy(ea_hbm.at[wid, j], ea_v)
        pltpu.sync_copy(rows_v, accx.at[dst_v], add=True)
        pltpu.sync_copy(ea_v, acce.at[dst_v], add=True)
        return carry

    lax.fori_loop(0, NCHUNK, body, 0)
    plsc.subcore_barrier()

    pltpu.sync_copy(accx.at[pl.ds(s * RPT, RPT)],
                    aggx_out.at[c, pl.ds(s * RPT, RPT)])
    pltpu.sync_copy(acce.at[pl.ds(s * RPT, RPT)],
                    agge_out.at[c, pl.ds(s * RPT, RPT)])


_sc_agg = pl.kernel(
    _sc_body,
    out_type=(
        jax.ShapeDtypeStruct((NC, N, D), jnp.float32),
        jax.ShapeDtypeStruct((NC, N, DE), jnp.float32),
    ),
    mesh=plsc.VectorSubcoreMesh(core_axis_name="c", subcore_axis_name="s"),
    scratch_types=[
        pltpu.VMEM((CH,), jnp.int32),
        pltpu.VMEM((CH,), jnp.int32),
        pltpu.VMEM((CH, D), jnp.float32),
        pltpu.VMEM((CH, DE), jnp.float32),
        pltpu.VMEM_SHARED((N, D), jnp.float32),
        pltpu.VMEM_SHARED((N, DE), jnp.float32),
        pltpu.SemaphoreType.DMA,
    ],
)


def _tc_body(x_r, aggx_r, agge_r, bat_r, ws_r, wm_r, we_r, bm_r, wp_r, bp_r,
             out_r, gsum_r, cnt_r):
    i = pl.program_id(0)

    @pl.when(i == 0)
    def _():
        gsum_r[...] = jnp.zeros_like(gsum_r)
        cnt_r[...] = jnp.zeros_like(cnt_r)

    x_b = x_r[...]
    aggx = aggx_r[0] + aggx_r[1]
    agge = agge_r[0] + agge_r[1]
    h = (jnp.dot(x_b, ws_r[...], preferred_element_type=jnp.float32)
         + jnp.dot(aggx, wm_r[...], preferred_element_type=jnp.float32)
         + jnp.dot(agge, we_r[...], preferred_element_type=jnp.float32)
         + bm_r[...])
    h = jnp.where(h > 0, h, 0.01 * h)

    bat = bat_r[...].reshape(1, BN)
    oh = (lax.broadcasted_iota(jnp.int32, (G, BN), 0) == bat
          ).astype(jnp.float32)
    gsum_r[...] += jnp.dot(oh, h, preferred_element_type=jnp.float32)
    cnt_r[...] += jnp.sum(oh, axis=1, keepdims=True)

    @pl.when(i == pl.num_programs(0) - 1)
    def _():
        gmean = gsum_r[...] / jnp.maximum(cnt_r[...], 1.0)
        nrm = jnp.sqrt(jnp.sum(gmean * gmean, axis=1, keepdims=True))
        embs = gmean / jnp.maximum(nrm, 1e-12)
        out_r[...] = (jnp.dot(embs, wp_r[...],
                              preferred_element_type=jnp.float32) + bp_r[...])


_tc_head = pl.pallas_call(
    _tc_body,
    grid=(NBLK,),
    in_specs=[
        pl.BlockSpec((BN, D), lambda i: (i, 0)),
        pl.BlockSpec((NC, BN, D), lambda i: (0, i, 0)),
        pl.BlockSpec((NC, BN, DE), lambda i: (0, i, 0)),
        pl.BlockSpec((1, 1, BN), lambda i: (i, 0, 0)),
        pl.BlockSpec((D, H), lambda i: (0, 0)),
        pl.BlockSpec((D, H), lambda i: (0, 0)),
        pl.BlockSpec((DE, H), lambda i: (0, 0)),
        pl.BlockSpec((1, H), lambda i: (0, 0)),
        pl.BlockSpec((H, DOUT), lambda i: (0, 0)),
        pl.BlockSpec((1, DOUT), lambda i: (0, 0)),
    ],
    out_specs=pl.BlockSpec((G, DOUT), lambda i: (0, 0)),
    out_shape=jax.ShapeDtypeStruct((G, DOUT), jnp.float32),
    scratch_shapes=[
        pltpu.VMEM((G, H), jnp.float32),
        pltpu.VMEM((G, 1), jnp.float32),
    ],
)


def kernel(x, edge_index, edge_attr, batch, W_self, W_msg, W_edge, b_msg,
           Wp, bp):
    ei = edge_index.reshape(2, NW, NCHUNK, CH)
    ea = edge_attr.reshape(NW, NCHUNK, CH, DE)
    z_d = jnp.zeros((N, D), jnp.float32)
    z_de = jnp.zeros((N, DE), jnp.float32)
    aggx2, agge2 = _sc_agg(z_d, z_de, x, ei, ea)
    bat3 = batch.reshape(NBLK, 1, BN)
    return _tc_head(x, aggx2, agge2, bat3, W_self, W_msg, W_edge,
                    b_msg.reshape(1, H), Wp, bp.reshape(1, DOUT))


# speedup vs baseline: 2.8229x; 2.8229x over previous
import jax
import jax.numpy as jnp
from jax import lax
from jax.experimental import pallas as pl
from jax.experimental.pallas import tpu as pltpu
from jax.experimental.pallas import tpu_sc as plsc

N = 10000
E = 320000
D = 128
DE = 16
H = 128
G = 64
DOUT = 1

NC = 2
NS = 16
NW = NC * NS
EPT = E // NW
CH = 80
NCHUNK = EPT // CH
NPAD = 10240
RPT = NPAD // NS

BN = 1000
NBLK = N // BN
EB = 4000
NEB = E // EB

HIGH = lax.Precision.HIGHEST


def _sc_body(z_d, y_hbm, m_hbm, src_hbm, dst_hbm, agg_out,
             src_v, dst_v, rows_v, mrows_v, acc, sem):
    c = lax.axis_index("c")
    s = lax.axis_index("s")
    wid = c * NS + s

    def init_body(r, carry):
        off = s * RPT + r * CH
        pltpu.sync_copy(z_d.at[pl.ds(off, CH)], rows_v)
        pltpu.sync_copy(rows_v, acc.at[pl.ds(off, CH)])
        return carry

    lax.fori_loop(0, RPT // CH, init_body, 0)
    plsc.subcore_barrier()

    def body(j, carry):
        base = wid * EPT + j * CH
        pltpu.sync_copy(src_hbm.at[pl.ds(base, CH)], src_v)
        pltpu.sync_copy(dst_hbm.at[pl.ds(base, CH)], dst_v)
        pltpu.async_copy(y_hbm.at[src_v], rows_v, sem).wait()
        pltpu.sync_copy(m_hbm.at[pl.ds(base, CH)], mrows_v)
        pltpu.sync_copy(rows_v, acc.at[dst_v], add=True)
        pltpu.sync_copy(mrows_v, acc.at[dst_v], add=True)
        return carry

    lax.fori_loop(0, NCHUNK, body, 0)
    plsc.subcore_barrier()

    def out_body(r, carry):
        off = s * RPT + r * CH
        pltpu.sync_copy(acc.at[pl.ds(off, CH)], rows_v)
        pltpu.sync_copy(rows_v, agg_out.at[c, pl.ds(off, CH)])
        return carry

    lax.fori_loop(0, RPT // CH, out_body, 0)


_sc_agg = pl.kernel(
    _sc_body,
    out_type=jax.ShapeDtypeStruct((NC, NPAD, D), jnp.float32),
    mesh=plsc.VectorSubcoreMesh(core_axis_name="c", subcore_axis_name="s"),
    scratch_types=[
        pltpu.VMEM((CH,), jnp.int32),
        pltpu.VMEM((CH,), jnp.int32),
        pltpu.VMEM((CH, D), jnp.float32),
        pltpu.VMEM((CH, D), jnp.float32),
        pltpu.VMEM_SHARED((NPAD, D), jnp.float32),
        pltpu.SemaphoreType.DMA,
    ],
)


def _mm_body(a_r, b_r, o_r):
    o_r[...] = jnp.dot(a_r[...], b_r[...], preferred_element_type=jnp.float32)


_y_mm = pl.pallas_call(
    _mm_body,
    grid=(NBLK,),
    in_specs=[
        pl.BlockSpec((BN, D), lambda i: (i, 0)),
        pl.BlockSpec((D, H), lambda i: (0, 0)),
    ],
    out_specs=pl.BlockSpec((BN, H), lambda i: (i, 0)),
    out_shape=jax.ShapeDtypeStruct((N, H), jnp.float32),
)

_m_mm = pl.pallas_call(
    _mm_body,
    grid=(NEB,),
    in_specs=[
        pl.BlockSpec((EB, DE), lambda i: (i, 0)),
        pl.BlockSpec((DE, H), lambda i: (0, 0)),
    ],
    out_specs=pl.BlockSpec((EB, H), lambda i: (i, 0)),
    out_shape=jax.ShapeDtypeStruct((E, H), jnp.float32),
)


def _tc_body(x_r, agg_r, bat_r, ws_r, bm_r, wp_r, bp_r,
             out_r, gsum_r, cnt_r):
    i = pl.program_id(0)

    @pl.when(i == 0)
    def _():
        gsum_r[...] = jnp.zeros_like(gsum_r)
        cnt_r[...] = jnp.zeros_like(cnt_r)

    h = (jnp.dot(x_r[...], ws_r[...], preferred_element_type=jnp.float32)
         + (agg_r[0] + agg_r[1]) + bm_r[...])
    h = jnp.where(h > 0, h, 0.01 * h)

    bat = bat_r[...].reshape(1, BN)
    oh = (lax.broadcasted_iota(jnp.int32, (G, BN), 0) == bat
          ).astype(jnp.float32)
    gsum_r[...] += jnp.dot(oh, h, preferred_element_type=jnp.float32,
                           precision=HIGH)
    cnt_r[...] += jnp.sum(oh, axis=1, keepdims=True)

    @pl.when(i == pl.num_programs(0) - 1)
    def _():
        gmean = gsum_r[...] / jnp.maximum(cnt_r[...], 1.0)
        nrm = jnp.sqrt(jnp.sum(gmean * gmean, axis=1, keepdims=True))
        embs = gmean / jnp.maximum(nrm, 1e-12)
        out_r[...] = (jnp.dot(embs, wp_r[...],
                              preferred_element_type=jnp.float32) + bp_r[...])


_tc_head = pl.pallas_call(
    _tc_body,
    grid=(NBLK,),
    in_specs=[
        pl.BlockSpec((BN, D), lambda i: (i, 0)),
        pl.BlockSpec((NC, BN, H), lambda i: (0, i, 0)),
        pl.BlockSpec((1, 1, BN), lambda i: (i, 0, 0)),
        pl.BlockSpec((D, H), lambda i: (0, 0)),
        pl.BlockSpec((1, H), lambda i: (0, 0)),
        pl.BlockSpec((H, DOUT), lambda i: (0, 0)),
        pl.BlockSpec((1, DOUT), lambda i: (0, 0)),
    ],
    out_specs=pl.BlockSpec((G, DOUT), lambda i: (0, 0)),
    out_shape=jax.ShapeDtypeStruct((G, DOUT), jnp.float32),
    scratch_shapes=[
        pltpu.VMEM((G, H), jnp.float32),
        pltpu.VMEM((G, 1), jnp.float32),
    ],
)


def kernel(x, edge_index, edge_attr, batch, W_self, W_msg, W_edge, b_msg,
           Wp, bp):
    src = edge_index[0]
    dst = edge_index[1]
    y = _y_mm(x, W_msg)
    m = _m_mm(edge_attr, W_edge)
    z_d = jnp.zeros((NPAD, D), jnp.float32)
    agg2 = _sc_agg(z_d, y, m, src, dst)
    bat3 = batch.reshape(NBLK, 1, BN)
    return _tc_head(x, agg2, bat3, W_self, b_msg.reshape(1, H),
                    Wp, bp.reshape(1, DOUT))


# speedup vs baseline: 3.9035x; 1.3828x over previous
import jax
import jax.numpy as jnp
from jax import lax
from jax.experimental import pallas as pl
from jax.experimental.pallas import tpu as pltpu
from jax.experimental.pallas import tpu_sc as plsc

N = 10000
E = 320000
D = 128
DE = 16
H = 128
G = 64
DOUT = 1

NC = 2
NS = 16
NW = NC * NS
EPT = E // NW
CH = 80
NCHUNK = EPT // CH
NPAD = 10240
RPT = NPAD // NS

BN = 1000
NBLK = N // BN
EB = 4000
NEB = E // EB

HIGH = lax.Precision.HIGHEST


def _sc_body(z_d, y_hbm, m_hbm, src_hbm, dst_hbm, agg_out,
             src_v0, dst_v0, rows_v0, mrows_v0,
             src_v1, dst_v1, rows_v1, mrows_v1,
             acc, sem_i, sem_g, sem_m):
    c = lax.axis_index("c")
    s = lax.axis_index("s")
    wid = c * NS + s

    def init_body(r, carry):
        off = s * RPT + r * CH
        pltpu.sync_copy(z_d.at[pl.ds(off, CH)], rows_v0)
        pltpu.sync_copy(rows_v0, acc.at[pl.ds(off, CH)])
        return carry

    lax.fori_loop(0, RPT // CH, init_body, 0)
    plsc.subcore_barrier()

    def body(t, carry):
        b0 = wid * EPT + (2 * t) * CH
        b1 = b0 + CH
        i0 = pltpu.async_copy(src_hbm.at[pl.ds(b0, CH)], src_v0, sem_i)
        i1 = pltpu.async_copy(dst_hbm.at[pl.ds(b0, CH)], dst_v0, sem_i)
        i2 = pltpu.async_copy(src_hbm.at[pl.ds(b1, CH)], src_v1, sem_i)
        i3 = pltpu.async_copy(dst_hbm.at[pl.ds(b1, CH)], dst_v1, sem_i)
        m0 = pltpu.async_copy(m_hbm.at[pl.ds(b0, CH)], mrows_v0, sem_m)
        m1 = pltpu.async_copy(m_hbm.at[pl.ds(b1, CH)], mrows_v1, sem_m)
        i0.wait(); i1.wait(); i2.wait(); i3.wait()
        g0 = pltpu.async_copy(y_hbm.at[src_v0], rows_v0, sem_g)
        g1 = pltpu.async_copy(y_hbm.at[src_v1], rows_v1, sem_g)
        g0.wait(); g1.wait(); m0.wait(); m1.wait()
        pltpu.sync_copy(rows_v0, acc.at[dst_v0], add=True)
        pltpu.sync_copy(mrows_v0, acc.at[dst_v0], add=True)
        pltpu.sync_copy(rows_v1, acc.at[dst_v1], add=True)
        pltpu.sync_copy(mrows_v1, acc.at[dst_v1], add=True)
        return carry

    lax.fori_loop(0, NCHUNK // 2, body, 0)

    bl = wid * EPT + (NCHUNK - 1) * CH
    pltpu.sync_copy(src_hbm.at[pl.ds(bl, CH)], src_v0)
    pltpu.sync_copy(dst_hbm.at[pl.ds(bl, CH)], dst_v0)
    pltpu.async_copy(y_hbm.at[src_v0], rows_v0, sem_g).wait()
    pltpu.sync_copy(m_hbm.at[pl.ds(bl, CH)], mrows_v0)
    pltpu.sync_copy(rows_v0, acc.at[dst_v0], add=True)
    pltpu.sync_copy(mrows_v0, acc.at[dst_v0], add=True)

    plsc.subcore_barrier()

    def out_body(r, carry):
        off = s * RPT + r * CH
        pltpu.sync_copy(acc.at[pl.ds(off, CH)], rows_v0)
        pltpu.sync_copy(rows_v0, agg_out.at[c, pl.ds(off, CH)])
        return carry

    lax.fori_loop(0, RPT // CH, out_body, 0)


_sc_agg = pl.kernel(
    _sc_body,
    out_type=jax.ShapeDtypeStruct((NC, NPAD, D), jnp.float32),
    mesh=plsc.VectorSubcoreMesh(core_axis_name="c", subcore_axis_name="s"),
    scratch_types=[
        pltpu.VMEM((CH,), jnp.int32),
        pltpu.VMEM((CH,), jnp.int32),
        pltpu.VMEM((CH, D), jnp.float32),
        pltpu.VMEM((CH, D), jnp.float32),
        pltpu.VMEM((CH,), jnp.int32),
        pltpu.VMEM((CH,), jnp.int32),
        pltpu.VMEM((CH, D), jnp.float32),
        pltpu.VMEM((CH, D), jnp.float32),
        pltpu.VMEM_SHARED((NPAD, D), jnp.float32),
        pltpu.SemaphoreType.DMA,
        pltpu.SemaphoreType.DMA,
        pltpu.SemaphoreType.DMA,
    ],
)


def _mm_body(a_r, b_r, o_r):
    o_r[...] = jnp.dot(a_r[...], b_r[...], preferred_element_type=jnp.float32)


_y_mm = pl.pallas_call(
    _mm_body,
    grid=(NBLK,),
    in_specs=[
        pl.BlockSpec((BN, D), lambda i: (i, 0)),
        pl.BlockSpec((D, H), lambda i: (0, 0)),
    ],
    out_specs=pl.BlockSpec((BN, H), lambda i: (i, 0)),
    out_shape=jax.ShapeDtypeStruct((N, H), jnp.float32),
)

_m_mm = pl.pallas_call(
    _mm_body,
    grid=(NEB,),
    in_specs=[
        pl.BlockSpec((EB, DE), lambda i: (i, 0)),
        pl.BlockSpec((DE, H), lambda i: (0, 0)),
    ],
    out_specs=pl.BlockSpec((EB, H), lambda i: (i, 0)),
    out_shape=jax.ShapeDtypeStruct((E, H), jnp.float32),
)


def _tc_body(x_r, agg_r, bat_r, ws_r, bm_r, wp_r, bp_r,
             out_r, gsum_r, cnt_r):
    i = pl.program_id(0)

    @pl.when(i == 0)
    def _():
        gsum_r[...] = jnp.zeros_like(gsum_r)
        cnt_r[...] = jnp.zeros_like(cnt_r)

    h = (jnp.dot(x_r[...], ws_r[...], preferred_element_type=jnp.float32)
         + (agg_r[0] + agg_r[1]) + bm_r[...])
    h = jnp.where(h > 0, h, 0.01 * h)

    bat = bat_r[...].reshape(1, BN)
    oh = (lax.broadcasted_iota(jnp.int32, (G, BN), 0) == bat
          ).astype(jnp.float32)
    gsum_r[...] += jnp.dot(oh, h, preferred_element_type=jnp.float32,
                           precision=HIGH)
    cnt_r[...] += jnp.sum(oh, axis=1, keepdims=True)

    @pl.when(i == pl.num_programs(0) - 1)
    def _():
        gmean = gsum_r[...] / jnp.maximum(cnt_r[...], 1.0)
        nrm = jnp.sqrt(jnp.sum(gmean * gmean, axis=1, keepdims=True))
        embs = gmean / jnp.maximum(nrm, 1e-12)
        out_r[...] = (jnp.dot(embs, wp_r[...],
                              preferred_element_type=jnp.float32) + bp_r[...])


_tc_head = pl.pallas_call(
    _tc_body,
    grid=(NBLK,),
    in_specs=[
        pl.BlockSpec((BN, D), lambda i: (i, 0)),
        pl.BlockSpec((NC, BN, H), lambda i: (0, i, 0)),
        pl.BlockSpec((1, 1, BN), lambda i: (i, 0, 0)),
        pl.BlockSpec((D, H), lambda i: (0, 0)),
        pl.BlockSpec((1, H), lambda i: (0, 0)),
        pl.BlockSpec((H, DOUT), lambda i: (0, 0)),
        pl.BlockSpec((1, DOUT), lambda i: (0, 0)),
    ],
    out_specs=pl.BlockSpec((G, DOUT), lambda i: (0, 0)),
    out_shape=jax.ShapeDtypeStruct((G, DOUT), jnp.float32),
    scratch_shapes=[
        pltpu.VMEM((G, H), jnp.float32),
        pltpu.VMEM((G, 1), jnp.float32),
    ],
)


def kernel(x, edge_index, edge_attr, batch, W_self, W_msg, W_edge, b_msg,
           Wp, bp):
    src = edge_index[0]
    dst = edge_index[1]
    y = _y_mm(x, W_msg)
    m = _m_mm(edge_attr, W_edge)
    z_d = jnp.zeros((NPAD, D), jnp.float32)
    agg2 = _sc_agg(z_d, y, m, src, dst)
    bat3 = batch.reshape(NBLK, 1, BN)
    return _tc_head(x, agg2, bat3, W_self, b_msg.reshape(1, H),
                    Wp, bp.reshape(1, DOUT))
